# initial kernel scaffold (unmeasured)
import jax
import jax.numpy as jnp
from jax import lax
from jax.experimental import pallas as pl
from jax.experimental.pallas import tpu as pltpu


def kernel(
    x,
):
    def body(*refs):
        pass

    out_shape = jax.ShapeDtypeStruct(..., jnp.float32)
    return pl.pallas_call(body, out_shape=out_shape)(...)


# baseline (device time: 81754 ns/iter reference)
import jax
import jax.numpy as jnp
from jax import lax
from jax.experimental import pallas as pl
from jax.experimental.pallas import tpu as pltpu

N_DEV = 4


def kernel(x):
    m_per, n = x.shape

    def body(x_ref, out_ref, comm_ref, send_sems, recv_sems):
        my_x = lax.axis_index("x")
        my_y = lax.axis_index("y")
        my_z = lax.axis_index("z")
        left = (my_y - 1) % N_DEV
        right = (my_y + 1) % N_DEV

        barrier_sem = pltpu.get_barrier_semaphore()
        for nbr in [left, right]:
            pl.semaphore_signal(
                barrier_sem, inc=1,
                device_id=(my_x, nbr, my_z),
                device_id_type=pl.DeviceIdType.MESH,
            )
        pl.semaphore_wait(barrier_sem, 2)

        out_ref[pl.ds(my_y * m_per, m_per), :] = x_ref[:, :]
        comm_ref[0, :, :] = x_ref[:, :]

        for h in range(N_DEV - 1):
            send_slot = h % 2
            recv_slot = (h + 1) % 2
            rdma = pltpu.make_async_remote_copy(
                src_ref=comm_ref.at[send_slot],
                dst_ref=comm_ref.at[recv_slot],
                send_sem=send_sems.at[send_slot],
                recv_sem=recv_sems.at[recv_slot],
                device_id=(my_x, right, my_z),
                device_id_type=pl.DeviceIdType.MESH,
            )
            rdma.start()
            rdma.wait()

            origin = (my_y - h - 1) % N_DEV
            out_ref[pl.ds(origin * m_per, m_per), :] = comm_ref[recv_slot, :, :]

    return pl.pallas_call(
        body,
        out_shape=jax.ShapeDtypeStruct((N_DEV * m_per, n), x.dtype),
        in_specs=[pl.BlockSpec(memory_space=pltpu.VMEM)],
        out_specs=pl.BlockSpec(memory_space=pltpu.VMEM),
        scratch_shapes=[
            pltpu.VMEM((2, m_per, n), x.dtype),
            pltpu.SemaphoreType.DMA((2,)),
            pltpu.SemaphoreType.DMA((2,)),
        ],
        compiler_params=pltpu.CompilerParams(collective_id=0),
    )(x)


# device time: 60332 ns/iter; 1.3551x vs baseline; 1.3551x over previous
import jax
import jax.numpy as jnp
from jax import lax
from jax.experimental import pallas as pl
from jax.experimental.pallas import tpu as pltpu

N_DEV = 4


def kernel(x):
    m_per, n = x.shape
    half = m_per // 2
    MESH = pl.DeviceIdType.MESH

    def body(x_ref, out_ref, r_send, r_recv, l_send, l_recv, xs_send, xs_recv):
        my_x = lax.axis_index("x")
        my_y = lax.axis_index("y")
        my_z = lax.axis_index("z")
        hb = my_x * half
        phb = (1 - my_x) * half

        right = (my_x, my_y + 1, my_z)
        left = (my_x, my_y - 1, my_z)
        partner = (1 - my_x, my_y, my_z)

        barrier = pltpu.get_barrier_semaphore()

        @pl.when(my_y > 0)
        def _():
            pl.semaphore_signal(barrier, inc=1, device_id=left,
                                device_id_type=MESH)

        @pl.when(my_y < N_DEV - 1)
        def _():
            pl.semaphore_signal(barrier, inc=1, device_id=right,
                                device_id_type=MESH)

        pl.semaphore_signal(barrier, inc=1, device_id=partner,
                            device_id_type=MESH)
        n_nbrs = (1 + (my_y > 0).astype(jnp.int32)
                  + (my_y < N_DEV - 1).astype(jnp.int32))
        pl.semaphore_wait(barrier, n_nbrs)

        def half_rows(origin, base):
            return pl.ds(origin * m_per + base, half)

        for s in range(N_DEV - 1):
            cond_rs = jnp.logical_and(my_y <= N_DEV - 2, my_y >= s)

            @pl.when(cond_rs)
            def _(s=s):
                origin = my_y - s
                src = (x_ref.at[pl.ds(hb, half), :] if s == 0
                       else out_ref.at[half_rows(origin, hb), :])
                rdma = pltpu.make_async_remote_copy(
                    src_ref=src,
                    dst_ref=out_ref.at[half_rows(origin, hb), :],
                    send_sem=r_send.at[s],
                    recv_sem=r_recv.at[s],
                    device_id=right,
                    device_id_type=MESH,
                )
                rdma.start()

            cond_ls = jnp.logical_and(my_y >= 1, my_y + s <= N_DEV - 1)

            @pl.when(cond_ls)
            def _(s=s):
                origin = my_y + s
                src = (x_ref.at[pl.ds(hb, half), :] if s == 0
                       else out_ref.at[half_rows(origin, hb), :])
                rdma = pltpu.make_async_remote_copy(
                    src_ref=src,
                    dst_ref=out_ref.at[half_rows(origin, hb), :],
                    send_sem=l_send.at[s],
                    recv_sem=l_recv.at[s],
                    device_id=left,
                    device_id_type=MESH,
                )
                rdma.start()

            if s == 0:
                out_ref[pl.ds(my_y * m_per, m_per), :] = x_ref[:, :]

            cond_rr = my_y >= s + 1

            @pl.when(cond_rr)
            def _(s=s):
                origin = my_y - 1 - s
                recv = pltpu.make_async_remote_copy(
                    src_ref=out_ref.at[half_rows(origin, hb), :],
                    dst_ref=out_ref.at[half_rows(origin, hb), :],
                    send_sem=r_send.at[s],
                    recv_sem=r_recv.at[s],
                    device_id=left,
                    device_id_type=MESH,
                )
                recv.wait_recv()
                fwd = pltpu.make_async_remote_copy(
                    src_ref=out_ref.at[half_rows(origin, hb), :],
                    dst_ref=out_ref.at[half_rows(origin, hb), :],
                    send_sem=xs_send.at[2 * s],
                    recv_sem=xs_recv.at[2 * s],
                    device_id=partner,
                    device_id_type=MESH,
                )
                fwd.start()

            cond_lr = my_y <= N_DEV - 2 - s

            @pl.when(cond_lr)
            def _(s=s):
                origin = my_y + 1 + s
                recv = pltpu.make_async_remote_copy(
                    src_ref=out_ref.at[half_rows(origin, hb), :],
                    dst_ref=out_ref.at[half_rows(origin, hb), :],
                    send_sem=l_send.at[s],
                    recv_sem=l_recv.at[s],
                    device_id=right,
                    device_id_type=MESH,
                )
                recv.wait_recv()
                fwd = pltpu.make_async_remote_copy(
                    src_ref=out_ref.at[half_rows(origin, hb), :],
                    dst_ref=out_ref.at[half_rows(origin, hb), :],
                    send_sem=xs_send.at[2 * s + 1],
                    recv_sem=xs_recv.at[2 * s + 1],
                    device_id=partner,
                    device_id_type=MESH,
                )
                fwd.start()

        for s in range(N_DEV - 1):
            @pl.when(my_y >= s + 1)
            def _(s=s):
                origin = my_y - 1 - s
                recv = pltpu.make_async_remote_copy(
                    src_ref=out_ref.at[half_rows(origin, phb), :],
                    dst_ref=out_ref.at[half_rows(origin, phb), :],
                    send_sem=xs_send.at[2 * s],
                    recv_sem=xs_recv.at[2 * s],
                    device_id=partner,
                    device_id_type=MESH,
                )
                recv.wait_recv()

            @pl.when(my_y <= N_DEV - 2 - s)
            def _(s=s):
                origin = my_y + 1 + s
                recv = pltpu.make_async_remote_copy(
                    src_ref=out_ref.at[half_rows(origin, phb), :],
                    dst_ref=out_ref.at[half_rows(origin, phb), :],
                    send_sem=xs_send.at[2 * s + 1],
                    recv_sem=xs_recv.at[2 * s + 1],
                    device_id=partner,
                    device_id_type=MESH,
                )
                recv.wait_recv()

        for s in range(N_DEV - 1):
            dummy = out_ref.at[half_rows(my_y, hb), :]

            @pl.when(jnp.logical_and(my_y <= N_DEV - 2, my_y >= s))
            def _(s=s, dummy=dummy):
                d = pltpu.make_async_remote_copy(
                    src_ref=dummy, dst_ref=dummy,
                    send_sem=r_send.at[s], recv_sem=r_recv.at[s],
                    device_id=right, device_id_type=MESH,
                )
                d.wait_send()

            @pl.when(jnp.logical_and(my_y >= 1, my_y + s <= N_DEV - 1))
            def _(s=s, dummy=dummy):
                d = pltpu.make_async_remote_copy(
                    src_ref=dummy, dst_ref=dummy,
                    send_sem=l_send.at[s], recv_sem=l_recv.at[s],
                    device_id=left, device_id_type=MESH,
                )
                d.wait_send()

            @pl.when(my_y >= s + 1)
            def _(s=s, dummy=dummy):
                d = pltpu.make_async_remote_copy(
                    src_ref=dummy, dst_ref=dummy,
                    send_sem=xs_send.at[2 * s], recv_sem=xs_recv.at[2 * s],
                    device_id=partner, device_id_type=MESH,
                )
                d.wait_send()

            @pl.when(my_y <= N_DEV - 2 - s)
            def _(s=s, dummy=dummy):
                d = pltpu.make_async_remote_copy(
                    src_ref=dummy, dst_ref=dummy,
                    send_sem=xs_send.at[2 * s + 1],
                    recv_sem=xs_recv.at[2 * s + 1],
                    device_id=partner, device_id_type=MESH,
                )
                d.wait_send()

    return pl.pallas_call(
        body,
        out_shape=jax.ShapeDtypeStruct((N_DEV * m_per, n), x.dtype),
        in_specs=[pl.BlockSpec(memory_space=pltpu.VMEM)],
        out_specs=pl.BlockSpec(memory_space=pltpu.VMEM),
        scratch_shapes=[
            pltpu.SemaphoreType.DMA((N_DEV - 1,)),
            pltpu.SemaphoreType.DMA((N_DEV - 1,)),
            pltpu.SemaphoreType.DMA((N_DEV - 1,)),
            pltpu.SemaphoreType.DMA((N_DEV - 1,)),
            pltpu.SemaphoreType.DMA((2 * (N_DEV - 1),)),
            pltpu.SemaphoreType.DMA((2 * (N_DEV - 1),)),
        ],
        compiler_params=pltpu.CompilerParams(collective_id=0),
    )(x)


# device time: 60316 ns/iter; 1.3554x vs baseline; 1.0003x over previous
import jax
import jax.numpy as jnp
from jax import lax
from jax.experimental import pallas as pl
from jax.experimental.pallas import tpu as pltpu

N_DEV = 4
SUB = 1


def kernel(x):
    m_per, n = x.shape
    half = m_per // 2
    sub = half // SUB
    NS = N_DEV - 1
    MESH = pl.DeviceIdType.MESH

    def body(x_ref, out_ref, r_send, r_recv, l_send, l_recv, xs_send, xs_recv):
        my_x = lax.axis_index("x")
        my_y = lax.axis_index("y")
        my_z = lax.axis_index("z")
        hb = my_x * half
        phb = (1 - my_x) * half

        right = (my_x, my_y + 1, my_z)
        left = (my_x, my_y - 1, my_z)
        partner = (1 - my_x, my_y, my_z)

        barrier = pltpu.get_barrier_semaphore()

        @pl.when(my_y > 0)
        def _():
            pl.semaphore_signal(barrier, inc=1, device_id=left,
                                device_id_type=MESH)

        @pl.when(my_y < N_DEV - 1)
        def _():
            pl.semaphore_signal(barrier, inc=1, device_id=right,
                                device_id_type=MESH)

        pl.semaphore_signal(barrier, inc=1, device_id=partner,
                            device_id_type=MESH)
        n_nbrs = (1 + (my_y > 0).astype(jnp.int32)
                  + (my_y < N_DEV - 1).astype(jnp.int32))
        pl.semaphore_wait(barrier, n_nbrs)

        def rows(origin, base, j):
            return pl.ds(origin * m_per + base + j * sub, sub)

        cond_rs = lambda s: jnp.logical_and(my_y <= N_DEV - 2, my_y >= s)
        cond_ls = lambda s: jnp.logical_and(my_y >= 1, my_y + s <= N_DEV - 1)
        cond_rr = lambda s: my_y >= s + 1
        cond_lr = lambda s: my_y <= N_DEV - 2 - s

        for s in range(NS):
            for j in range(SUB):
                k = s * SUB + j

                @pl.when(cond_rs(s))
                def _(s=s, j=j, k=k):
                    origin = my_y - s
                    src = (x_ref.at[pl.ds(hb + j * sub, sub), :] if s == 0
                           else out_ref.at[rows(origin, hb, j), :])
                    pltpu.make_async_remote_copy(
                        src_ref=src,
                        dst_ref=out_ref.at[rows(origin, hb, j), :],
                        send_sem=r_send.at[k], recv_sem=r_recv.at[k],
                        device_id=right, device_id_type=MESH,
                    ).start()

                @pl.when(cond_ls(s))
                def _(s=s, j=j, k=k):
                    origin = my_y + s
                    src = (x_ref.at[pl.ds(hb + j * sub, sub), :] if s == 0
                           else out_ref.at[rows(origin, hb, j), :])
                    pltpu.make_async_remote_copy(
                        src_ref=src,
                        dst_ref=out_ref.at[rows(origin, hb, j), :],
                        send_sem=l_send.at[k], recv_sem=l_recv.at[k],
                        device_id=left, device_id_type=MESH,
                    ).start()

            if s == 0:
                out_ref[pl.ds(my_y * m_per, m_per), :] = x_ref[:, :]

            for j in range(SUB):
                k = s * SUB + j

                @pl.when(cond_rr(s))
                def _(s=s, j=j, k=k):
                    origin = my_y - 1 - s
                    dst = out_ref.at[rows(origin, hb, j), :]
                    pltpu.make_async_remote_copy(
                        src_ref=dst, dst_ref=dst,
                        send_sem=r_send.at[k], recv_sem=r_recv.at[k],
                        device_id=left, device_id_type=MESH,
                    ).wait_recv()
                    pltpu.make_async_remote_copy(
                        src_ref=dst, dst_ref=dst,
                        send_sem=xs_send.at[2 * k], recv_sem=xs_recv.at[2 * k],
                        device_id=partner, device_id_type=MESH,
                    ).start()

                @pl.when(cond_lr(s))
                def _(s=s, j=j, k=k):
                    origin = my_y + 1 + s
                    dst = out_ref.at[rows(origin, hb, j), :]
                    pltpu.make_async_remote_copy(
                        src_ref=dst, dst_ref=dst,
                        send_sem=l_send.at[k], recv_sem=l_recv.at[k],
                        device_id=right, device_id_type=MESH,
                    ).wait_recv()
                    pltpu.make_async_remote_copy(
                        src_ref=dst, dst_ref=dst,
                        send_sem=xs_send.at[2 * k + 1],
                        recv_sem=xs_recv.at[2 * k + 1],
                        device_id=partner, device_id_type=MESH,
                    ).start()

        for s in range(NS):
            for j in range(SUB):
                k = s * SUB + j

                @pl.when(cond_rr(s))
                def _(s=s, j=j, k=k):
                    dst = out_ref.at[rows(my_y - 1 - s, phb, j), :]
                    pltpu.make_async_remote_copy(
                        src_ref=dst, dst_ref=dst,
                        send_sem=xs_send.at[2 * k], recv_sem=xs_recv.at[2 * k],
                        device_id=partner, device_id_type=MESH,
                    ).wait_recv()

                @pl.when(cond_lr(s))
                def _(s=s, j=j, k=k):
                    dst = out_ref.at[rows(my_y + 1 + s, phb, j), :]
                    pltpu.make_async_remote_copy(
                        src_ref=dst, dst_ref=dst,
                        send_sem=xs_send.at[2 * k + 1],
                        recv_sem=xs_recv.at[2 * k + 1],
                        device_id=partner, device_id_type=MESH,
                    ).wait_recv()

        dummy = out_ref.at[pl.ds(my_y * m_per, sub), :]
        for s in range(NS):
            for j in range(SUB):
                k = s * SUB + j

                @pl.when(cond_rs(s))
                def _(k=k):
                    pltpu.make_async_remote_copy(
                        src_ref=dummy, dst_ref=dummy,
                        send_sem=r_send.at[k], recv_sem=r_recv.at[k],
                        device_id=right, device_id_type=MESH,
                    ).wait_send()

                @pl.when(cond_ls(s))
                def _(k=k):
                    pltpu.make_async_remote_copy(
                        src_ref=dummy, dst_ref=dummy,
                        send_sem=l_send.at[k], recv_sem=l_recv.at[k],
                        device_id=left, device_id_type=MESH,
                    ).wait_send()

                @pl.when(cond_rr(s))
                def _(k=k):
                    pltpu.make_async_remote_copy(
                        src_ref=dummy, dst_ref=dummy,
                        send_sem=xs_send.at[2 * k], recv_sem=xs_recv.at[2 * k],
                        device_id=partner, device_id_type=MESH,
                    ).wait_send()

                @pl.when(cond_lr(s))
                def _(k=k):
                    pltpu.make_async_remote_copy(
                        src_ref=dummy, dst_ref=dummy,
                        send_sem=xs_send.at[2 * k + 1],
                        recv_sem=xs_recv.at[2 * k + 1],
                        device_id=partner, device_id_type=MESH,
                    ).wait_send()

    nsem = (N_DEV - 1) * SUB
    return pl.pallas_call(
        body,
        out_shape=jax.ShapeDtypeStruct((N_DEV * m_per, n), x.dtype),
        in_specs=[pl.BlockSpec(memory_space=pltpu.VMEM)],
        out_specs=pl.BlockSpec(memory_space=pltpu.VMEM),
        scratch_shapes=[
            pltpu.SemaphoreType.DMA((nsem,)),
            pltpu.SemaphoreType.DMA((nsem,)),
            pltpu.SemaphoreType.DMA((nsem,)),
            pltpu.SemaphoreType.DMA((nsem,)),
            pltpu.SemaphoreType.DMA((2 * nsem,)),
            pltpu.SemaphoreType.DMA((2 * nsem,)),
        ],
        compiler_params=pltpu.CompilerParams(collective_id=0),
    )(x)


# device time: 54908 ns/iter; 1.4889x vs baseline; 1.0985x over previous
import jax
import jax.numpy as jnp
from jax import lax
from jax.experimental import pallas as pl
from jax.experimental.pallas import tpu as pltpu

N_DEV = 4
SUB = 2


def kernel(x):
    m_per, n = x.shape
    half = m_per // 2
    sub = half // SUB
    NS = N_DEV - 1
    MESH = pl.DeviceIdType.MESH

    def body(x_ref, out_ref, r_send, r_recv, l_send, l_recv, xs_send, xs_recv):
        my_x = lax.axis_index("x")
        my_y = lax.axis_index("y")
        my_z = lax.axis_index("z")
        hb = my_x * half
        phb = (1 - my_x) * half

        right = (my_x, my_y + 1, my_z)
        left = (my_x, my_y - 1, my_z)
        partner = (1 - my_x, my_y, my_z)

        barrier = pltpu.get_barrier_semaphore()

        @pl.when(my_y > 0)
        def _():
            pl.semaphore_signal(barrier, inc=1, device_id=left,
                                device_id_type=MESH)

        @pl.when(my_y < N_DEV - 1)
        def _():
            pl.semaphore_signal(barrier, inc=1, device_id=right,
                                device_id_type=MESH)

        pl.semaphore_signal(barrier, inc=1, device_id=partner,
                            device_id_type=MESH)
        n_nbrs = (1 + (my_y > 0).astype(jnp.int32)
                  + (my_y < N_DEV - 1).astype(jnp.int32))
        pl.semaphore_wait(barrier, n_nbrs)

        def rows(origin, base, j):
            return pl.ds(origin * m_per + base + j * sub, sub)

        cond_rs = lambda s: jnp.logical_and(my_y <= N_DEV - 2, my_y >= s)
        cond_ls = lambda s: jnp.logical_and(my_y >= 1, my_y + s <= N_DEV - 1)
        cond_rr = lambda s: my_y >= s + 1
        cond_lr = lambda s: my_y <= N_DEV - 2 - s

        for s in range(NS):
            for j in range(SUB):
                k = s * SUB + j

                @pl.when(cond_rs(s))
                def _(s=s, j=j, k=k):
                    origin = my_y - s
                    src = (x_ref.at[pl.ds(hb + j * sub, sub), :] if s == 0
                           else out_ref.at[rows(origin, hb, j), :])
                    pltpu.make_async_remote_copy(
                        src_ref=src,
                        dst_ref=out_ref.at[rows(origin, hb, j), :],
                        send_sem=r_send.at[k], recv_sem=r_recv.at[k],
                        device_id=right, device_id_type=MESH,
                    ).start()

                @pl.when(cond_ls(s))
                def _(s=s, j=j, k=k):
                    origin = my_y + s
                    src = (x_ref.at[pl.ds(hb + j * sub, sub), :] if s == 0
                           else out_ref.at[rows(origin, hb, j), :])
                    pltpu.make_async_remote_copy(
                        src_ref=src,
                        dst_ref=out_ref.at[rows(origin, hb, j), :],
                        send_sem=l_send.at[k], recv_sem=l_recv.at[k],
                        device_id=left, device_id_type=MESH,
                    ).start()

            if s == 0:
                out_ref[pl.ds(my_y * m_per, m_per), :] = x_ref[:, :]

            for j in range(SUB):
                k = s * SUB + j

                @pl.when(cond_rr(s))
                def _(s=s, j=j, k=k):
                    origin = my_y - 1 - s
                    dst = out_ref.at[rows(origin, hb, j), :]
                    pltpu.make_async_remote_copy(
                        src_ref=dst, dst_ref=dst,
                        send_sem=r_send.at[k], recv_sem=r_recv.at[k],
                        device_id=left, device_id_type=MESH,
                    ).wait_recv()
                    pltpu.make_async_remote_copy(
                        src_ref=dst, dst_ref=dst,
                        send_sem=xs_send.at[2 * k], recv_sem=xs_recv.at[2 * k],
                        device_id=partner, device_id_type=MESH,
                    ).start()

                @pl.when(cond_lr(s))
                def _(s=s, j=j, k=k):
                    origin = my_y + 1 + s
                    dst = out_ref.at[rows(origin, hb, j), :]
                    pltpu.make_async_remote_copy(
                        src_ref=dst, dst_ref=dst,
                        send_sem=l_send.at[k], recv_sem=l_recv.at[k],
                        device_id=right, device_id_type=MESH,
                    ).wait_recv()
                    pltpu.make_async_remote_copy(
                        src_ref=dst, dst_ref=dst,
                        send_sem=xs_send.at[2 * k + 1],
                        recv_sem=xs_recv.at[2 * k + 1],
                        device_id=partner, device_id_type=MESH,
                    ).start()

        for s in range(NS):
            for j in range(SUB):
                k = s * SUB + j

                @pl.when(cond_rr(s))
                def _(s=s, j=j, k=k):
                    dst = out_ref.at[rows(my_y - 1 - s, phb, j), :]
                    pltpu.make_async_remote_copy(
                        src_ref=dst, dst_ref=dst,
                        send_sem=xs_send.at[2 * k], recv_sem=xs_recv.at[2 * k],
                        device_id=partner, device_id_type=MESH,
                    ).wait_recv()

                @pl.when(cond_lr(s))
                def _(s=s, j=j, k=k):
                    dst = out_ref.at[rows(my_y + 1 + s, phb, j), :]
                    pltpu.make_async_remote_copy(
                        src_ref=dst, dst_ref=dst,
                        send_sem=xs_send.at[2 * k + 1],
                        recv_sem=xs_recv.at[2 * k + 1],
                        device_id=partner, device_id_type=MESH,
                    ).wait_recv()

        dummy = out_ref.at[pl.ds(my_y * m_per, sub), :]
        for s in range(NS):
            for j in range(SUB):
                k = s * SUB + j

                @pl.when(cond_rs(s))
                def _(k=k):
                    pltpu.make_async_remote_copy(
                        src_ref=dummy, dst_ref=dummy,
                        send_sem=r_send.at[k], recv_sem=r_recv.at[k],
                        device_id=right, device_id_type=MESH,
                    ).wait_send()

                @pl.when(cond_ls(s))
                def _(k=k):
                    pltpu.make_async_remote_copy(
                        src_ref=dummy, dst_ref=dummy,
                        send_sem=l_send.at[k], recv_sem=l_recv.at[k],
                        device_id=left, device_id_type=MESH,
                    ).wait_send()

                @pl.when(cond_rr(s))
                def _(k=k):
                    pltpu.make_async_remote_copy(
                        src_ref=dummy, dst_ref=dummy,
                        send_sem=xs_send.at[2 * k], recv_sem=xs_recv.at[2 * k],
                        device_id=partner, device_id_type=MESH,
                    ).wait_send()

                @pl.when(cond_lr(s))
                def _(k=k):
                    pltpu.make_async_remote_copy(
                        src_ref=dummy, dst_ref=dummy,
                        send_sem=xs_send.at[2 * k + 1],
                        recv_sem=xs_recv.at[2 * k + 1],
                        device_id=partner, device_id_type=MESH,
                    ).wait_send()

    nsem = (N_DEV - 1) * SUB
    return pl.pallas_call(
        body,
        out_shape=jax.ShapeDtypeStruct((N_DEV * m_per, n), x.dtype),
        in_specs=[pl.BlockSpec(memory_space=pltpu.VMEM)],
        out_specs=pl.BlockSpec(memory_space=pltpu.VMEM),
        scratch_shapes=[
            pltpu.SemaphoreType.DMA((nsem,)),
            pltpu.SemaphoreType.DMA((nsem,)),
            pltpu.SemaphoreType.DMA((nsem,)),
            pltpu.SemaphoreType.DMA((nsem,)),
            pltpu.SemaphoreType.DMA((2 * nsem,)),
            pltpu.SemaphoreType.DMA((2 * nsem,)),
        ],
        compiler_params=pltpu.CompilerParams(collective_id=0),
    )(x)


# device time: 52631 ns/iter; 1.5533x vs baseline; 1.0433x over previous
import jax
import jax.numpy as jnp
from jax import lax
from jax.experimental import pallas as pl
from jax.experimental.pallas import tpu as pltpu

N_DEV = 4
SUB = 4


def kernel(x):
    m_per, n = x.shape
    half = m_per // 2
    sub = half // SUB
    NS = N_DEV - 1
    MESH = pl.DeviceIdType.MESH

    def body(x_ref, out_ref, r_send, r_recv, l_send, l_recv, xs_send, xs_recv):
        my_x = lax.axis_index("x")
        my_y = lax.axis_index("y")
        my_z = lax.axis_index("z")
        hb = my_x * half
        phb = (1 - my_x) * half

        right = (my_x, my_y + 1, my_z)
        left = (my_x, my_y - 1, my_z)
        partner = (1 - my_x, my_y, my_z)

        barrier = pltpu.get_barrier_semaphore()

        @pl.when(my_y > 0)
        def _():
            pl.semaphore_signal(barrier, inc=1, device_id=left,
                                device_id_type=MESH)

        @pl.when(my_y < N_DEV - 1)
        def _():
            pl.semaphore_signal(barrier, inc=1, device_id=right,
                                device_id_type=MESH)

        pl.semaphore_signal(barrier, inc=1, device_id=partner,
                            device_id_type=MESH)
        n_nbrs = (1 + (my_y > 0).astype(jnp.int32)
                  + (my_y < N_DEV - 1).astype(jnp.int32))
        pl.semaphore_wait(barrier, n_nbrs)

        def rows(origin, base, j):
            return pl.ds(origin * m_per + base + j * sub, sub)

        cond_rs = lambda s: jnp.logical_and(my_y <= N_DEV - 2, my_y >= s)
        cond_ls = lambda s: jnp.logical_and(my_y >= 1, my_y + s <= N_DEV - 1)
        cond_rr = lambda s: my_y >= s + 1
        cond_lr = lambda s: my_y <= N_DEV - 2 - s

        for s in range(NS):
            for j in range(SUB):
                k = s * SUB + j

                @pl.when(cond_rs(s))
                def _(s=s, j=j, k=k):
                    origin = my_y - s
                    src = (x_ref.at[pl.ds(hb + j * sub, sub), :] if s == 0
                           else out_ref.at[rows(origin, hb, j), :])
                    pltpu.make_async_remote_copy(
                        src_ref=src,
                        dst_ref=out_ref.at[rows(origin, hb, j), :],
                        send_sem=r_send.at[k], recv_sem=r_recv.at[k],
                        device_id=right, device_id_type=MESH,
                    ).start()

                @pl.when(cond_ls(s))
                def _(s=s, j=j, k=k):
                    origin = my_y + s
                    src = (x_ref.at[pl.ds(hb + j * sub, sub), :] if s == 0
                           else out_ref.at[rows(origin, hb, j), :])
                    pltpu.make_async_remote_copy(
                        src_ref=src,
                        dst_ref=out_ref.at[rows(origin, hb, j), :],
                        send_sem=l_send.at[k], recv_sem=l_recv.at[k],
                        device_id=left, device_id_type=MESH,
                    ).start()

            if s == 0:
                out_ref[pl.ds(my_y * m_per, m_per), :] = x_ref[:, :]

            for j in range(SUB):
                k = s * SUB + j

                @pl.when(cond_rr(s))
                def _(s=s, j=j, k=k):
                    origin = my_y - 1 - s
                    dst = out_ref.at[rows(origin, hb, j), :]
                    pltpu.make_async_remote_copy(
                        src_ref=dst, dst_ref=dst,
                        send_sem=r_send.at[k], recv_sem=r_recv.at[k],
                        device_id=left, device_id_type=MESH,
                    ).wait_recv()
                    pltpu.make_async_remote_copy(
                        src_ref=dst, dst_ref=dst,
                        send_sem=xs_send.at[2 * k], recv_sem=xs_recv.at[2 * k],
                        device_id=partner, device_id_type=MESH,
                    ).start()

                @pl.when(cond_lr(s))
                def _(s=s, j=j, k=k):
                    origin = my_y + 1 + s
                    dst = out_ref.at[rows(origin, hb, j), :]
                    pltpu.make_async_remote_copy(
                        src_ref=dst, dst_ref=dst,
                        send_sem=l_send.at[k], recv_sem=l_recv.at[k],
                        device_id=right, device_id_type=MESH,
                    ).wait_recv()
                    pltpu.make_async_remote_copy(
                        src_ref=dst, dst_ref=dst,
                        send_sem=xs_send.at[2 * k + 1],
                        recv_sem=xs_recv.at[2 * k + 1],
                        device_id=partner, device_id_type=MESH,
                    ).start()

        for s in range(NS):
            for j in range(SUB):
                k = s * SUB + j

                @pl.when(cond_rr(s))
                def _(s=s, j=j, k=k):
                    dst = out_ref.at[rows(my_y - 1 - s, phb, j), :]
                    pltpu.make_async_remote_copy(
                        src_ref=dst, dst_ref=dst,
                        send_sem=xs_send.at[2 * k], recv_sem=xs_recv.at[2 * k],
                        device_id=partner, device_id_type=MESH,
                    ).wait_recv()

                @pl.when(cond_lr(s))
                def _(s=s, j=j, k=k):
                    dst = out_ref.at[rows(my_y + 1 + s, phb, j), :]
                    pltpu.make_async_remote_copy(
                        src_ref=dst, dst_ref=dst,
                        send_sem=xs_send.at[2 * k + 1],
                        recv_sem=xs_recv.at[2 * k + 1],
                        device_id=partner, device_id_type=MESH,
                    ).wait_recv()

        dummy = out_ref.at[pl.ds(my_y * m_per, sub), :]
        for s in range(NS):
            for j in range(SUB):
                k = s * SUB + j

                @pl.when(cond_rs(s))
                def _(k=k):
                    pltpu.make_async_remote_copy(
                        src_ref=dummy, dst_ref=dummy,
                        send_sem=r_send.at[k], recv_sem=r_recv.at[k],
                        device_id=right, device_id_type=MESH,
                    ).wait_send()

                @pl.when(cond_ls(s))
                def _(k=k):
                    pltpu.make_async_remote_copy(
                        src_ref=dummy, dst_ref=dummy,
                        send_sem=l_send.at[k], recv_sem=l_recv.at[k],
                        device_id=left, device_id_type=MESH,
                    ).wait_send()

                @pl.when(cond_rr(s))
                def _(k=k):
                    pltpu.make_async_remote_copy(
                        src_ref=dummy, dst_ref=dummy,
                        send_sem=xs_send.at[2 * k], recv_sem=xs_recv.at[2 * k],
                        device_id=partner, device_id_type=MESH,
                    ).wait_send()

                @pl.when(cond_lr(s))
                def _(k=k):
                    pltpu.make_async_remote_copy(
                        src_ref=dummy, dst_ref=dummy,
                        send_sem=xs_send.at[2 * k + 1],
                        recv_sem=xs_recv.at[2 * k + 1],
                        device_id=partner, device_id_type=MESH,
                    ).wait_send()

    nsem = (N_DEV - 1) * SUB
    return pl.pallas_call(
        body,
        out_shape=jax.ShapeDtypeStruct((N_DEV * m_per, n), x.dtype),
        in_specs=[pl.BlockSpec(memory_space=pltpu.VMEM)],
        out_specs=pl.BlockSpec(memory_space=pltpu.VMEM),
        scratch_shapes=[
            pltpu.SemaphoreType.DMA((nsem,)),
            pltpu.SemaphoreType.DMA((nsem,)),
            pltpu.SemaphoreType.DMA((nsem,)),
            pltpu.SemaphoreType.DMA((nsem,)),
            pltpu.SemaphoreType.DMA((2 * nsem,)),
            pltpu.SemaphoreType.DMA((2 * nsem,)),
        ],
        compiler_params=pltpu.CompilerParams(collective_id=0),
    )(x)
